# baseline (speedup 1.0000x reference)
import jax
import jax.numpy as jnp
from jax import lax
from jax.experimental import pallas as pl
from jax.experimental.pallas import tpu as pltpu

_B, _N, _E, _H = 8, 2048, 128, 128
_KP, _W = 5, 11


def _decode_kernel(nodes_ref, tgt_ref, y_W_ref, y_b_ref, h_W_ref, h_b_ref,
                   Wx_ref, bx_ref, Wh_ref, bh_ref, att_Wh_ref, att_Wn_ref,
                   wc_ref, vblk_ref, words_ref, loss_ref, projT_ref, t16_ref,
                   gxn_ref):
    f32 = jnp.float32
    bf16 = jnp.bfloat16
    B, N, E, H = _B, _N, _E, _H

    att_Wn = att_Wn_ref[...]
    Wx = Wx_ref[...]
    for b in range(B):
        projT_ref[b] = lax.dot_general(
            att_Wn, nodes_ref[b],
            dimension_numbers=(((0,), (1,)), ((), ())),
            preferred_element_type=f32)
        gxn_ref[b] = jnp.dot(nodes_ref[b], Wx, preferred_element_type=f32)

    ctx_rows = [jnp.sum(nodes_ref[b], axis=0, keepdims=True) * (1.0 / N)
                for b in range(B)]
    context = jnp.concatenate(ctx_rows, axis=0)

    y_W = y_W_ref[...]
    y_b = y_b_ref[...]
    h_W = h_W_ref[...]
    h_b = h_b_ref[...]
    bx = bx_ref[...]
    Wh = Wh_ref[...]
    bh = bh_ref[...]
    att_Wh = att_Wh_ref[...]
    wc_col = wc_ref[...]
    vblk16 = vblk_ref[...]

    iota2 = lax.broadcasted_iota(jnp.int32, (B, N), 1)

    def word_step(l, w, h, gx, used, kp_sum, cnt, loss):
        step = l * _W + w
        gh = jnp.dot(h, Wh, preferred_element_type=f32) + bh
        z = jax.nn.sigmoid(gx[:, 0:H] + gh[:, 0:H])
        r = jax.nn.sigmoid(gx[:, H:2 * H] + gh[:, H:2 * H])
        n = jnp.tanh(gx[:, 2 * H:3 * H] + r * gh[:, 2 * H:3 * H])
        h = (1.0 - z) * n + z * h

        hpT = lax.dot_general(att_Wh, h,
                              dimension_numbers=(((0,), (1,)), ((), ())),
                              preferred_element_type=f32)

        for b in range(B):
            t16_ref[b * H:(b + 1) * H, :] = jnp.tanh(
                projT_ref[b] + hpT[:, b:b + 1]).astype(bf16)
        e_all = jnp.dot(vblk16, t16_ref[...],
                        preferred_element_type=f32)
        m = jnp.max(e_all, axis=1, keepdims=True)
        idx_col = jnp.min(jnp.where(e_all == m, iota2, N),
                          axis=1, keepdims=True)
        oh = (iota2 == idx_col).astype(f32)
        new_mask = 1.0 - jnp.sum(oh * used, axis=1, keepdims=True)
        used = jnp.maximum(used, oh)
        lse = m + jnp.log(jnp.sum(jnp.exp(e_all - m), axis=1, keepdims=True))
        tgt_col = jnp.concatenate(
            [jnp.full((1, 1), tgt_ref[step, b], jnp.int32) for b in range(B)],
            axis=0)
        et = jnp.sum(jnp.where(iota2 == tgt_col, e_all, 0.0),
                     axis=1, keepdims=True)
        loss = loss + jnp.sum(lse - et) * (1.0 / B)
        inp_rows = []
        gx_rows = []
        for b in range(B):
            idx_b = idx_col[b, 0]
            words_ref[step, b] = idx_b
            inp_rows.append(nodes_ref[b, pl.ds(idx_b, 1), :])
            gx_rows.append(gxn_ref[b, pl.ds(idx_b, 1), :])
            base_b = pl.multiple_of((idx_b // 128) * 128, 128)
            lmask = (lax.broadcasted_iota(jnp.int32, (1, 128), 1)
                     == idx_b % 128).astype(f32)
            projT_ref[b, :, pl.ds(base_b, 128)] = (
                projT_ref[b, :, pl.ds(base_b, 128)] + wc_col * lmask)
        inp = jnp.concatenate(inp_rows, axis=0)
        gx = jnp.concatenate(gx_rows, axis=0) + bx
        kp_sum = kp_sum + new_mask * inp
        cnt = cnt + new_mask
        return h, gx, used, kp_sum, cnt, loss

    used = jnp.zeros((B, N), f32)
    kp_sum = jnp.zeros((B, E), f32)
    cnt = jnp.zeros((B, 1), f32)
    loss = jnp.float32(0.0)
    h = jnp.zeros((B, H), f32)

    for l in range(_KP):
        if l == 0:
            gx = jnp.broadcast_to(bx, (B, 3 * H))
            h = jnp.dot(context, h_W, preferred_element_type=f32) + h_b
        else:
            kp_mean = kp_sum / jnp.maximum(cnt, 1.0)
            cy = context * kp_mean
            inp = jnp.dot(cy, y_W, preferred_element_type=f32) + y_b
            gx = jnp.dot(inp, Wx, preferred_element_type=f32) + bx
            h = jnp.dot(cy, h_W, preferred_element_type=f32) + h_b

        def body(w, carry):
            h, gx, used, kp_sum, cnt, loss = carry
            return word_step(l, w, h, gx, used, kp_sum, cnt, loss)

        h, gx, used, kp_sum, cnt, loss = lax.fori_loop(
            0, _W, body, (h, gx, used, kp_sum, cnt, loss))

    loss_ref[0, 0] = loss


@jax.jit
def kernel(nodes, targets, last_ids, y_W, y_b, h_W, h_b, gru_Wx, gru_Wh,
           gru_bx, gru_bh, att_Wn, att_Wh, att_wc, att_v):
    del last_ids
    f32 = jnp.float32
    tgt2d = targets.astype(jnp.int32).reshape(_KP * _W, _B)
    vblk = jnp.kron(jnp.eye(_B, dtype=f32),
                    att_v.reshape(1, _H)).astype(jnp.bfloat16)

    words, loss = pl.pallas_call(
        _decode_kernel,
        out_shape=[
            jax.ShapeDtypeStruct((_KP * _W, _B), jnp.int32),
            jax.ShapeDtypeStruct((1, 1), f32),
        ],
        in_specs=[
            pl.BlockSpec(memory_space=pltpu.VMEM),
            pl.BlockSpec(memory_space=pltpu.SMEM),
        ] + [pl.BlockSpec(memory_space=pltpu.VMEM)] * 12,
        out_specs=[
            pl.BlockSpec(memory_space=pltpu.SMEM),
            pl.BlockSpec(memory_space=pltpu.SMEM),
        ],
        scratch_shapes=[pltpu.VMEM((_B, _H, _N), f32),
                        pltpu.VMEM((_B * _H, _N), jnp.bfloat16),
                        pltpu.VMEM((_B, _N, 3 * _H), f32)],
    )(nodes, tgt2d,
      y_W, y_b.reshape(1, _E), h_W, h_b.reshape(1, _H),
      gru_Wx, gru_bx.reshape(1, 3 * _H), gru_Wh, gru_bh.reshape(1, 3 * _H),
      att_Wh, att_Wn, att_wc.reshape(_H, 1), vblk)

    keyphrases = words.reshape(_KP, _W, _B).transpose(0, 2, 1)
    return keyphrases, loss[0, 0]


# speedup vs baseline: 1.0284x; 1.0284x over previous
import jax
import jax.numpy as jnp
from jax import lax
from jax.experimental import pallas as pl
from jax.experimental.pallas import tpu as pltpu

_B, _N, _E, _H = 8, 2048, 128, 128
_KP, _W = 5, 11


def _decode_kernel(nodes_ref, tgt_ref, y_W_ref, y_b_ref, h_W_ref, h_b_ref,
                   Wx_ref, bx_ref, Wh_ref, bh_ref, att_Wh_ref, att_Wn_ref,
                   wc_ref, vblk_ref, words_ref, loss_ref, projT_ref, t16_ref):
    f32 = jnp.float32
    bf16 = jnp.bfloat16
    B, N, E, H = _B, _N, _E, _H

    att_Wn = att_Wn_ref[...]
    Wx = Wx_ref[...]
    for b in range(B):
        projT_ref[b] = lax.dot_general(
            att_Wn, nodes_ref[b],
            dimension_numbers=(((0,), (1,)), ((), ())),
            preferred_element_type=f32)

    ctx_rows = [jnp.sum(nodes_ref[b], axis=0, keepdims=True) * (1.0 / N)
                for b in range(B)]
    context = jnp.concatenate(ctx_rows, axis=0)

    y_W = y_W_ref[...]
    y_b = y_b_ref[...]
    h_W = h_W_ref[...]
    h_b = h_b_ref[...]
    bx = bx_ref[...]
    Wh = Wh_ref[...]
    bh = bh_ref[...]
    att_Wh = att_Wh_ref[...]
    wc_col = wc_ref[...]
    vblk16 = vblk_ref[...]

    iota2 = lax.broadcasted_iota(jnp.int32, (B, N), 1)

    def word_step(l, w, h, inp, used, kp_sum, cnt, loss):
        step = l * _W + w
        gx = jnp.dot(inp, Wx, preferred_element_type=f32) + bx
        gh = jnp.dot(h, Wh, preferred_element_type=f32) + bh
        z = jax.nn.sigmoid(gx[:, 0:H] + gh[:, 0:H])
        r = jax.nn.sigmoid(gx[:, H:2 * H] + gh[:, H:2 * H])
        n = jnp.tanh(gx[:, 2 * H:3 * H] + r * gh[:, 2 * H:3 * H])
        h = (1.0 - z) * n + z * h

        hpT = lax.dot_general(att_Wh, h,
                              dimension_numbers=(((0,), (1,)), ((), ())),
                              preferred_element_type=f32)

        for b in range(B):
            t16_ref[b * H:(b + 1) * H, :] = jnp.tanh(
                projT_ref[b] + hpT[:, b:b + 1]).astype(bf16)
        e_all = jnp.dot(vblk16, t16_ref[...],
                        preferred_element_type=f32)
        m = jnp.max(e_all, axis=1, keepdims=True)
        idx_col = jnp.min(jnp.where(e_all == m, iota2, N),
                          axis=1, keepdims=True)
        oh = (iota2 == idx_col).astype(f32)
        new_mask = 1.0 - jnp.sum(oh * used, axis=1, keepdims=True)
        used = jnp.maximum(used, oh)
        lse = m + jnp.log(jnp.sum(jnp.exp(e_all - m), axis=1, keepdims=True))
        tgt_col = jnp.concatenate(
            [jnp.full((1, 1), tgt_ref[step, b], jnp.int32) for b in range(B)],
            axis=0)
        et = jnp.sum(jnp.where(iota2 == tgt_col, e_all, 0.0),
                     axis=1, keepdims=True)
        loss = loss + jnp.sum(lse - et) * (1.0 / B)
        inp_rows = []
        for b in range(B):
            idx_b = idx_col[b, 0]
            words_ref[step, b] = idx_b
            inp_rows.append(nodes_ref[b, pl.ds(idx_b, 1), :])
            base_b = pl.multiple_of((idx_b // 128) * 128, 128)
            lmask = (lax.broadcasted_iota(jnp.int32, (1, 128), 1)
                     == idx_b % 128).astype(f32)
            projT_ref[b, :, pl.ds(base_b, 128)] = (
                projT_ref[b, :, pl.ds(base_b, 128)] + wc_col * lmask)
        inp = jnp.concatenate(inp_rows, axis=0)
        kp_sum = kp_sum + new_mask * inp
        cnt = cnt + new_mask
        return h, inp, used, kp_sum, cnt, loss

    used = jnp.zeros((B, N), f32)
    kp_sum = jnp.zeros((B, E), f32)
    cnt = jnp.zeros((B, 1), f32)
    loss = jnp.float32(0.0)
    h = jnp.zeros((B, H), f32)

    for l in range(_KP):
        if l == 0:
            inp = jnp.zeros((B, E), f32)
            h = jnp.dot(context, h_W, preferred_element_type=f32) + h_b
        else:
            kp_mean = kp_sum / jnp.maximum(cnt, 1.0)
            cy = context * kp_mean
            inp = jnp.dot(cy, y_W, preferred_element_type=f32) + y_b
            h = jnp.dot(cy, h_W, preferred_element_type=f32) + h_b

        def body(w, carry):
            h, inp, used, kp_sum, cnt, loss = carry
            return word_step(l, w, h, inp, used, kp_sum, cnt, loss)

        h, inp, used, kp_sum, cnt, loss = lax.fori_loop(
            0, _W, body, (h, inp, used, kp_sum, cnt, loss))

    loss_ref[0, 0] = loss


@jax.jit
def kernel(nodes, targets, last_ids, y_W, y_b, h_W, h_b, gru_Wx, gru_Wh,
           gru_bx, gru_bh, att_Wn, att_Wh, att_wc, att_v):
    del last_ids
    f32 = jnp.float32
    tgt2d = targets.astype(jnp.int32).reshape(_KP * _W, _B)
    vblk = jnp.kron(jnp.eye(_B, dtype=f32),
                    att_v.reshape(1, _H)).astype(jnp.bfloat16)

    words, loss = pl.pallas_call(
        _decode_kernel,
        out_shape=[
            jax.ShapeDtypeStruct((_KP * _W, _B), jnp.int32),
            jax.ShapeDtypeStruct((1, 1), f32),
        ],
        in_specs=[
            pl.BlockSpec(memory_space=pltpu.VMEM),
            pl.BlockSpec(memory_space=pltpu.SMEM),
        ] + [pl.BlockSpec(memory_space=pltpu.VMEM)] * 12,
        out_specs=[
            pl.BlockSpec(memory_space=pltpu.SMEM),
            pl.BlockSpec(memory_space=pltpu.SMEM),
        ],
        scratch_shapes=[pltpu.VMEM((_B, _H, _N), f32),
                        pltpu.VMEM((_B * _H, _N), jnp.bfloat16)],
    )(nodes, tgt2d,
      y_W, y_b.reshape(1, _E), h_W, h_b.reshape(1, _H),
      gru_Wx, gru_bx.reshape(1, 3 * _H), gru_Wh, gru_bh.reshape(1, 3 * _H),
      att_Wh, att_Wn, att_wc.reshape(_H, 1), vblk)

    keyphrases = words.reshape(_KP, _W, _B).transpose(0, 2, 1)
    return keyphrases, loss[0, 0]


# speedup vs baseline: 1.0330x; 1.0044x over previous
import jax
import jax.numpy as jnp
from jax import lax
from jax.experimental import pallas as pl
from jax.experimental.pallas import tpu as pltpu

_B, _N, _E, _H = 8, 2048, 128, 128
_KP, _W = 5, 11


def _decode_kernel(nodes_ref, tgt_ref, y_W_ref, y_b_ref, h_W_ref, h_b_ref,
                   Wx_ref, bx_ref, Wh_ref, bh_ref, att_Wh_ref, att_Wn_ref,
                   wc_ref, vblk_ref, words_ref, loss_ref, t16_ref,
                   *projT_refs):
    f32 = jnp.float32
    bf16 = jnp.bfloat16
    B, N, E, H = _B, _N, _E, _H

    att_Wn = att_Wn_ref[...]
    Wx = Wx_ref[...]
    for b in range(B):
        projT_refs[b][...] = lax.dot_general(
            att_Wn, nodes_ref[b],
            dimension_numbers=(((0,), (1,)), ((), ())),
            preferred_element_type=f32)

    ctx_rows = [jnp.sum(nodes_ref[b], axis=0, keepdims=True) * (1.0 / N)
                for b in range(B)]
    context = jnp.concatenate(ctx_rows, axis=0)

    y_W = y_W_ref[...]
    y_b = y_b_ref[...]
    h_W = h_W_ref[...]
    h_b = h_b_ref[...]
    bx = bx_ref[...]
    Wh = Wh_ref[...]
    bh = bh_ref[...]
    att_Wh = att_Wh_ref[...]
    wc_col = wc_ref[...]
    vblk16 = vblk_ref[...]

    iota2 = lax.broadcasted_iota(jnp.int32, (B, N), 1)

    def word_step(l, w, h, inp, used, kp_sum, cnt, loss):
        step = l * _W + w
        gx = jnp.dot(inp, Wx, preferred_element_type=f32) + bx
        gh = jnp.dot(h, Wh, preferred_element_type=f32) + bh
        z = jax.nn.sigmoid(gx[:, 0:H] + gh[:, 0:H])
        r = jax.nn.sigmoid(gx[:, H:2 * H] + gh[:, H:2 * H])
        n = jnp.tanh(gx[:, 2 * H:3 * H] + r * gh[:, 2 * H:3 * H])
        h = (1.0 - z) * n + z * h

        hpT = lax.dot_general(att_Wh, h,
                              dimension_numbers=(((0,), (1,)), ((), ())),
                              preferred_element_type=f32)

        for b in range(B):
            t16_ref[b * H:(b + 1) * H, :] = jnp.tanh(
                projT_refs[b][...] + hpT[:, b:b + 1]).astype(bf16)
        e_all = jnp.dot(vblk16, t16_ref[...],
                        preferred_element_type=f32)
        m = jnp.max(e_all, axis=1, keepdims=True)
        idx_col = jnp.min(jnp.where(e_all == m, iota2, N),
                          axis=1, keepdims=True)
        oh = (iota2 == idx_col).astype(f32)
        new_mask = 1.0 - jnp.sum(oh * used, axis=1, keepdims=True)
        used = jnp.maximum(used, oh)
        lse = m + jnp.log(jnp.sum(jnp.exp(e_all - m), axis=1, keepdims=True))
        tgt_col = jnp.concatenate(
            [jnp.full((1, 1), tgt_ref[step, b], jnp.int32) for b in range(B)],
            axis=0)
        et = jnp.sum(jnp.where(iota2 == tgt_col, e_all, 0.0),
                     axis=1, keepdims=True)
        loss = loss + jnp.sum(lse - et) * (1.0 / B)
        inp_rows = []
        for b in range(B):
            idx_b = idx_col[b, 0]
            words_ref[step, b] = idx_b
            inp_rows.append(nodes_ref[b, pl.ds(idx_b, 1), :])
            base_b = pl.multiple_of((idx_b // 128) * 128, 128)
            lmask = (lax.broadcasted_iota(jnp.int32, (1, 128), 1)
                     == idx_b % 128).astype(f32)
            projT_refs[b][:, pl.ds(base_b, 128)] = (
                projT_refs[b][:, pl.ds(base_b, 128)] + wc_col * lmask)
        inp = jnp.concatenate(inp_rows, axis=0)
        kp_sum = kp_sum + new_mask * inp
        cnt = cnt + new_mask
        return h, inp, used, kp_sum, cnt, loss

    used = jnp.zeros((B, N), f32)
    kp_sum = jnp.zeros((B, E), f32)
    cnt = jnp.zeros((B, 1), f32)
    loss = jnp.float32(0.0)
    h = jnp.zeros((B, H), f32)

    for l in range(_KP):
        if l == 0:
            inp = jnp.zeros((B, E), f32)
            h = jnp.dot(context, h_W, preferred_element_type=f32) + h_b
        else:
            kp_mean = kp_sum / jnp.maximum(cnt, 1.0)
            cy = context * kp_mean
            inp = jnp.dot(cy, y_W, preferred_element_type=f32) + y_b
            h = jnp.dot(cy, h_W, preferred_element_type=f32) + h_b

        def body(w, carry):
            h, inp, used, kp_sum, cnt, loss = carry
            return word_step(l, w, h, inp, used, kp_sum, cnt, loss)

        h, inp, used, kp_sum, cnt, loss = lax.fori_loop(
            0, _W, body, (h, inp, used, kp_sum, cnt, loss))

    loss_ref[0, 0] = loss


@jax.jit
def kernel(nodes, targets, last_ids, y_W, y_b, h_W, h_b, gru_Wx, gru_Wh,
           gru_bx, gru_bh, att_Wn, att_Wh, att_wc, att_v):
    del last_ids
    f32 = jnp.float32
    tgt2d = targets.astype(jnp.int32).reshape(_KP * _W, _B)
    vblk = jnp.kron(jnp.eye(_B, dtype=f32),
                    att_v.reshape(1, _H)).astype(jnp.bfloat16)

    words, loss = pl.pallas_call(
        _decode_kernel,
        out_shape=[
            jax.ShapeDtypeStruct((_KP * _W, _B), jnp.int32),
            jax.ShapeDtypeStruct((1, 1), f32),
        ],
        in_specs=[
            pl.BlockSpec(memory_space=pltpu.VMEM),
            pl.BlockSpec(memory_space=pltpu.SMEM),
        ] + [pl.BlockSpec(memory_space=pltpu.VMEM)] * 12,
        out_specs=[
            pl.BlockSpec(memory_space=pltpu.SMEM),
            pl.BlockSpec(memory_space=pltpu.SMEM),
        ],
        scratch_shapes=[pltpu.VMEM((_B * _H, _N), jnp.bfloat16)]
                       + [pltpu.VMEM((_H, _N), f32) for _ in range(_B)],
    )(nodes, tgt2d,
      y_W, y_b.reshape(1, _E), h_W, h_b.reshape(1, _H),
      gru_Wx, gru_bx.reshape(1, 3 * _H), gru_Wh, gru_bh.reshape(1, 3 * _H),
      att_Wh, att_Wn, att_wc.reshape(_H, 1), vblk)

    keyphrases = words.reshape(_KP, _W, _B).transpose(0, 2, 1)
    return keyphrases, loss[0, 0]


# speedup vs baseline: 1.0374x; 1.0043x over previous
import jax
import jax.numpy as jnp
from jax import lax
from jax.experimental import pallas as pl
from jax.experimental.pallas import tpu as pltpu

_B, _N, _E, _H = 8, 2048, 128, 128
_KP, _W = 5, 11


def _decode_kernel(nodes_ref, tgt_ref, y_W_ref, y_b_ref, h_W_ref, h_b_ref,
                   Wx_ref, bx_ref, Wh_ref, bh_ref, att_Wh_ref, att_Wn_ref,
                   wc_ref, vblk_ref, words_ref, loss_ref, t16_ref,
                   *projT_refs):
    f32 = jnp.float32
    bf16 = jnp.bfloat16
    B, N, E, H = _B, _N, _E, _H

    att_Wn = att_Wn_ref[...]
    Wx = Wx_ref[...]
    for b in range(B):
        projT_refs[b][...] = lax.dot_general(
            att_Wn, nodes_ref[b],
            dimension_numbers=(((0,), (1,)), ((), ())),
            preferred_element_type=f32)

    ctx_rows = [jnp.sum(nodes_ref[b], axis=0, keepdims=True) * (1.0 / N)
                for b in range(B)]
    context = jnp.concatenate(ctx_rows, axis=0)

    y_W = y_W_ref[...]
    y_b = y_b_ref[...]
    h_W = h_W_ref[...]
    h_b = h_b_ref[...]
    bx = bx_ref[...]
    Wh = Wh_ref[...]
    bh = bh_ref[...]
    att_Wh = att_Wh_ref[...]
    wc_col = wc_ref[...]
    vblk16 = vblk_ref[...]

    iota2 = lax.broadcasted_iota(jnp.int32, (B, N), 1)

    def word_step(l, w, h, inp, used, kp_sum, cnt, loss):
        step = l * _W + w
        gx = jnp.dot(inp, Wx, preferred_element_type=f32) + bx
        gh = jnp.dot(h, Wh, preferred_element_type=f32) + bh
        z = jax.nn.sigmoid(gx[:, 0:H] + gh[:, 0:H])
        r = jax.nn.sigmoid(gx[:, H:2 * H] + gh[:, H:2 * H])
        n = jnp.tanh(gx[:, 2 * H:3 * H] + r * gh[:, 2 * H:3 * H])
        h = (1.0 - z) * n + z * h

        hpT = lax.dot_general(att_Wh, h,
                              dimension_numbers=(((0,), (1,)), ((), ())),
                              preferred_element_type=f32)

        t_rows = [jnp.tanh(projT_refs[b][...] + hpT[:, b:b + 1]).astype(bf16)
                  for b in range(B)]
        e_all = jnp.dot(vblk16, jnp.concatenate(t_rows, axis=0),
                        preferred_element_type=f32)
        m = jnp.max(e_all, axis=1, keepdims=True)
        idx_col = jnp.min(jnp.where(e_all == m, iota2, N),
                          axis=1, keepdims=True)
        oh = (iota2 == idx_col).astype(f32)
        new_mask = 1.0 - jnp.sum(oh * used, axis=1, keepdims=True)
        used = jnp.maximum(used, oh)
        lse = m + jnp.log(jnp.sum(jnp.exp(e_all - m), axis=1, keepdims=True))
        tgt_col = jnp.concatenate(
            [jnp.full((1, 1), tgt_ref[step, b], jnp.int32) for b in range(B)],
            axis=0)
        et = jnp.sum(jnp.where(iota2 == tgt_col, e_all, 0.0),
                     axis=1, keepdims=True)
        loss = loss + jnp.sum(lse - et) * (1.0 / B)
        inp_rows = []
        for b in range(B):
            idx_b = idx_col[b, 0]
            words_ref[step, b] = idx_b
            inp_rows.append(nodes_ref[b, pl.ds(idx_b, 1), :])
            base_b = pl.multiple_of((idx_b // 128) * 128, 128)
            lmask = (lax.broadcasted_iota(jnp.int32, (1, 128), 1)
                     == idx_b % 128).astype(f32)
            projT_refs[b][:, pl.ds(base_b, 128)] = (
                projT_refs[b][:, pl.ds(base_b, 128)] + wc_col * lmask)
        inp = jnp.concatenate(inp_rows, axis=0)
        kp_sum = kp_sum + new_mask * inp
        cnt = cnt + new_mask
        return h, inp, used, kp_sum, cnt, loss

    used = jnp.zeros((B, N), f32)
    kp_sum = jnp.zeros((B, E), f32)
    cnt = jnp.zeros((B, 1), f32)
    loss = jnp.float32(0.0)
    h = jnp.zeros((B, H), f32)

    for l in range(_KP):
        if l == 0:
            inp = jnp.zeros((B, E), f32)
            h = jnp.dot(context, h_W, preferred_element_type=f32) + h_b
        else:
            kp_mean = kp_sum / jnp.maximum(cnt, 1.0)
            cy = context * kp_mean
            inp = jnp.dot(cy, y_W, preferred_element_type=f32) + y_b
            h = jnp.dot(cy, h_W, preferred_element_type=f32) + h_b

        def body(w, carry):
            h, inp, used, kp_sum, cnt, loss = carry
            return word_step(l, w, h, inp, used, kp_sum, cnt, loss)

        h, inp, used, kp_sum, cnt, loss = lax.fori_loop(
            0, _W, body, (h, inp, used, kp_sum, cnt, loss))

    loss_ref[0, 0] = loss


@jax.jit
def kernel(nodes, targets, last_ids, y_W, y_b, h_W, h_b, gru_Wx, gru_Wh,
           gru_bx, gru_bh, att_Wn, att_Wh, att_wc, att_v):
    del last_ids
    f32 = jnp.float32
    tgt2d = targets.astype(jnp.int32).reshape(_KP * _W, _B)
    vblk = jnp.kron(jnp.eye(_B, dtype=f32),
                    att_v.reshape(1, _H)).astype(jnp.bfloat16)

    words, loss = pl.pallas_call(
        _decode_kernel,
        out_shape=[
            jax.ShapeDtypeStruct((_KP * _W, _B), jnp.int32),
            jax.ShapeDtypeStruct((1, 1), f32),
        ],
        in_specs=[
            pl.BlockSpec(memory_space=pltpu.VMEM),
            pl.BlockSpec(memory_space=pltpu.SMEM),
        ] + [pl.BlockSpec(memory_space=pltpu.VMEM)] * 12,
        out_specs=[
            pl.BlockSpec(memory_space=pltpu.SMEM),
            pl.BlockSpec(memory_space=pltpu.SMEM),
        ],
        scratch_shapes=[pltpu.VMEM((_B * _H, _N), jnp.bfloat16)]
                       + [pltpu.VMEM((_H, _N), f32) for _ in range(_B)],
    )(nodes, tgt2d,
      y_W, y_b.reshape(1, _E), h_W, h_b.reshape(1, _H),
      gru_Wx, gru_bx.reshape(1, 3 * _H), gru_Wh, gru_bh.reshape(1, 3 * _H),
      att_Wh, att_Wn, att_wc.reshape(_H, 1), vblk)

    keyphrases = words.reshape(_KP, _W, _B).transpose(0, 2, 1)
    return keyphrases, loss[0, 0]


# speedup vs baseline: 1.0381x; 1.0007x over previous
import jax
import jax.numpy as jnp
from jax import lax
from jax.experimental import pallas as pl
from jax.experimental.pallas import tpu as pltpu

_B, _N, _E, _H = 8, 2048, 128, 128
_KP, _W = 5, 11


def _decode_kernel(nodes_ref, tgt_ref, y_W_ref, y_b_ref, h_W_ref, h_b_ref,
                   Wx_ref, bx_ref, Wh_ref, bh_ref, att_Wh_ref, att_Wn_ref,
                   wc_ref, vblk_ref, words_ref, loss_ref, *projT_refs):
    f32 = jnp.float32
    bf16 = jnp.bfloat16
    B, N, E, H = _B, _N, _E, _H

    att_Wn = att_Wn_ref[...]
    Wx = Wx_ref[...]
    for b in range(B):
        projT_refs[b][...] = lax.dot_general(
            att_Wn, nodes_ref[b],
            dimension_numbers=(((0,), (1,)), ((), ())),
            preferred_element_type=f32)

    ctx_rows = [jnp.sum(nodes_ref[b], axis=0, keepdims=True) * (1.0 / N)
                for b in range(B)]
    context = jnp.concatenate(ctx_rows, axis=0)

    y_W = y_W_ref[...]
    y_b = y_b_ref[...]
    h_W = h_W_ref[...]
    h_b = h_b_ref[...]
    bx = bx_ref[...]
    Wh = Wh_ref[...]
    bh = bh_ref[...]
    att_Wh = att_Wh_ref[...]
    wc_col = wc_ref[...]
    vblk16 = vblk_ref[...]

    iota2 = lax.broadcasted_iota(jnp.int32, (B, N), 1)

    def word_step(l, w, h, inp, used, kp_sum, cnt, loss):
        step = l * _W + w
        gx = jnp.dot(inp, Wx, preferred_element_type=f32) + bx
        gh = jnp.dot(h, Wh, preferred_element_type=f32) + bh
        z = jax.nn.sigmoid(gx[:, 0:H] + gh[:, 0:H])
        r = jax.nn.sigmoid(gx[:, H:2 * H] + gh[:, H:2 * H])
        n = jnp.tanh(gx[:, 2 * H:3 * H] + r * gh[:, 2 * H:3 * H])
        h = (1.0 - z) * n + z * h

        hpT = lax.dot_general(att_Wh, h,
                              dimension_numbers=(((0,), (1,)), ((), ())),
                              preferred_element_type=f32)

        t_rows = [jnp.tanh(projT_refs[b][...] + hpT[:, b:b + 1]).astype(bf16)
                  for b in range(B)]
        e_all = jnp.dot(vblk16, jnp.concatenate(t_rows, axis=0),
                        preferred_element_type=f32)
        m = jnp.max(e_all, axis=1, keepdims=True)
        idx_col = jnp.min(jnp.where(e_all == m, iota2, N),
                          axis=1, keepdims=True)
        oh = (iota2 == idx_col).astype(f32)
        new_mask = 1.0 - jnp.sum(oh * used, axis=1, keepdims=True)
        used = jnp.maximum(used, oh)
        lse = m + jnp.log(jnp.sum(jnp.exp(e_all - m), axis=1, keepdims=True))
        tgt_col = jnp.concatenate(
            [jnp.full((1, 1), tgt_ref[step, b], jnp.int32) for b in range(B)],
            axis=0)
        et = jnp.sum(jnp.where(iota2 == tgt_col, e_all, 0.0),
                     axis=1, keepdims=True)
        loss = loss + jnp.sum(lse - et) * (1.0 / B)
        inp_rows = []
        for b in range(B):
            idx_b = idx_col[b, 0]
            words_ref[step, b] = idx_b
            inp_rows.append(nodes_ref[b, pl.ds(idx_b, 1), :])
            base_b = pl.multiple_of((idx_b // 128) * 128, 128)
            lmask = (lax.broadcasted_iota(jnp.int32, (1, 128), 1)
                     == idx_b % 128).astype(f32)
            projT_refs[b][:, pl.ds(base_b, 128)] = (
                projT_refs[b][:, pl.ds(base_b, 128)] + wc_col * lmask)
        inp = jnp.concatenate(inp_rows, axis=0)
        kp_sum = kp_sum + new_mask * inp
        cnt = cnt + new_mask
        return h, inp, used, kp_sum, cnt, loss

    used = jnp.zeros((B, N), f32)
    kp_sum = jnp.zeros((B, E), f32)
    cnt = jnp.zeros((B, 1), f32)
    loss = jnp.float32(0.0)
    h = jnp.zeros((B, H), f32)

    for l in range(_KP):
        if l == 0:
            inp = jnp.zeros((B, E), f32)
            h = jnp.dot(context, h_W, preferred_element_type=f32) + h_b
        else:
            kp_mean = kp_sum / jnp.maximum(cnt, 1.0)
            cy = context * kp_mean
            inp = jnp.dot(cy, y_W, preferred_element_type=f32) + y_b
            h = jnp.dot(cy, h_W, preferred_element_type=f32) + h_b

        def body(w, carry):
            h, inp, used, kp_sum, cnt, loss = carry
            return word_step(l, w, h, inp, used, kp_sum, cnt, loss)

        h, inp, used, kp_sum, cnt, loss = lax.fori_loop(
            0, _W, body, (h, inp, used, kp_sum, cnt, loss))

    loss_ref[0, 0] = loss


@jax.jit
def kernel(nodes, targets, last_ids, y_W, y_b, h_W, h_b, gru_Wx, gru_Wh,
           gru_bx, gru_bh, att_Wn, att_Wh, att_wc, att_v):
    del last_ids
    f32 = jnp.float32
    tgt2d = targets.astype(jnp.int32).reshape(_KP * _W, _B)
    vblk = jnp.kron(jnp.eye(_B, dtype=f32),
                    att_v.reshape(1, _H)).astype(jnp.bfloat16)

    words, loss = pl.pallas_call(
        _decode_kernel,
        out_shape=[
            jax.ShapeDtypeStruct((_KP * _W, _B), jnp.int32),
            jax.ShapeDtypeStruct((1, 1), f32),
        ],
        in_specs=[
            pl.BlockSpec(memory_space=pltpu.VMEM),
            pl.BlockSpec(memory_space=pltpu.SMEM),
        ] + [pl.BlockSpec(memory_space=pltpu.VMEM)] * 12,
        out_specs=[
            pl.BlockSpec(memory_space=pltpu.SMEM),
            pl.BlockSpec(memory_space=pltpu.SMEM),
        ],
        scratch_shapes=[pltpu.VMEM((_H, _N), f32) for _ in range(_B)],
    )(nodes, tgt2d,
      y_W, y_b.reshape(1, _E), h_W, h_b.reshape(1, _H),
      gru_Wx, gru_bx.reshape(1, 3 * _H), gru_Wh, gru_bh.reshape(1, 3 * _H),
      att_Wh, att_Wn, att_wc.reshape(_H, 1), vblk)

    keyphrases = words.reshape(_KP, _W, _B).transpose(0, 2, 1)
    return keyphrases, loss[0, 0]
